# baseline (speedup 1.0000x reference)
import functools

import jax
import jax.numpy as jnp
from jax import lax
from jax.experimental import pallas as pl
from jax.experimental.pallas import tpu as pltpu
from jax.experimental.pallas import tpu_sc as plsc

N = 8192
D = 512
E = 8
K = 2
A = N * K
TILE = 512
P = A + E * TILE
NT = P // TILE
NTP = 48
GT = 1024
LANES = 128
CT = 256
SUB = 64


def _gate_body(xt_ref, gw_ref, gb_ref,
               i1_ref, i2_ref, p1_ref, p2_ref, cnt_ref, xf_ref):
    xb = xt_ref[...]
    logits = lax.dot_general(gw_ref[...], xb, (((1,), (0,)), ((), ())),
                             preferred_element_type=jnp.float32,
                             precision=jax.lax.Precision.HIGHEST) + gb_ref[...]
    sub = jax.lax.broadcasted_iota(jnp.int32, (LANES, GT), 0)
    neg = jnp.float32(-1e30)
    l1 = jnp.where(sub < E, logits, neg)
    m1 = jnp.max(l1, axis=0)
    i1 = jnp.min(jnp.where(l1 == m1[None, :], sub, LANES), axis=0)
    l2 = jnp.where(sub == i1[None, :], neg, l1)
    m2 = jnp.max(l2, axis=0)
    i2 = jnp.min(jnp.where(l2 == m2[None, :], sub, LANES), axis=0)
    p1 = 1.0 / (1.0 + jnp.exp(m2 - m1))
    i1_ref[0, 0, :] = i1
    i2_ref[0, 0, :] = i2
    p1_ref[0, 0, :] = p1
    p2_ref[0, 0, :] = 1.0 - p1
    oh = ((sub == i1[None, :]).astype(jnp.int32)
          + (sub == i2[None, :]).astype(jnp.int32))
    for q in range(GT // CT):
        cnt_ref[0, q, :] = jnp.sum(oh[:, q * CT:(q + 1) * CT], axis=1)
    xf_ref[...] = xb.T


def _gate(xt, gate_w, gate_b):
    gwp = jnp.zeros((LANES, D), jnp.float32).at[:E].set(gate_w)
    gbp = jnp.zeros((LANES, 1), jnp.float32).at[:E, 0].set(gate_b)
    ng = N // GT
    nq = GT // CT
    outs = pl.pallas_call(
        _gate_body,
        grid=(ng,),
        in_specs=[
            pl.BlockSpec((D, GT), lambda i: (0, i)),
            pl.BlockSpec((LANES, D), lambda i: (0, 0)),
            pl.BlockSpec((LANES, 1), lambda i: (0, 0)),
        ],
        out_specs=[pl.BlockSpec((1, 1, GT), lambda i: (i, 0, 0))] * 4
        + [pl.BlockSpec((1, nq, LANES), lambda i: (i, 0, 0)),
           pl.BlockSpec((GT, D), lambda i: (i, 0))],
        out_shape=[
            jax.ShapeDtypeStruct((ng, 1, GT), jnp.int32),
            jax.ShapeDtypeStruct((ng, 1, GT), jnp.int32),
            jax.ShapeDtypeStruct((ng, 1, GT), jnp.float32),
            jax.ShapeDtypeStruct((ng, 1, GT), jnp.float32),
            jax.ShapeDtypeStruct((ng, nq, LANES), jnp.int32),
            jax.ShapeDtypeStruct((N, D), jnp.float32),
        ],
    )(xt, gwp, gbp)
    i1, i2, p1, p2 = (o.reshape(N) for o in outs[:4])
    counts = outs[4].reshape(N // CT, LANES)
    return i1, i2, p1, p2, counts, outs[5]


@functools.lru_cache(maxsize=1)
def _make_dispatch():
    mesh = plsc.VectorSubcoreMesh(core_axis_name="c", subcore_axis_name="s")
    return functools.partial(
        pl.kernel,
        mesh=mesh,
        compiler_params=pltpu.CompilerParams(needs_layout_passes=False),
        out_type=[
            jax.ShapeDtypeStruct((N // CT, CT // SUB, SUB), jnp.int32),
            jax.ShapeDtypeStruct((N // CT, CT // SUB, SUB), jnp.int32),
            jax.ShapeDtypeStruct((NTP,), jnp.int32),
            jax.ShapeDtypeStruct((P, D), jnp.float32),
        ],
        scratch_types=[
            pltpu.VMEM((N // CT, LANES), jnp.int32),
            pltpu.VMEM((CT,), jnp.int32),
            pltpu.VMEM((CT,), jnp.int32),
            pltpu.VMEM((CT // SUB, SUB), jnp.int32),
            pltpu.VMEM((CT // SUB, SUB), jnp.int32),
            pltpu.VMEM((NTP,), jnp.int32),
            pltpu.VMEM((2, SUB, D), jnp.float32),
            pltpu.SemaphoreType.DMA,
            pltpu.SemaphoreType.DMA,
        ],
    )(_dispatch_body)


def _dispatch_body(i1_hbm, i2_hbm, cnts_hbm, xf_hbm,
                   d0_hbm, d1_hbm, texp_hbm, xs_hbm,
                   cnt_v, i1_v, i2_v, d0_v, d1_v, texp_v, xin_v,
                   sem_a, sem_b):
    c = lax.axis_index("c")
    s = lax.axis_index("s")
    w = s * 2 + c
    sem_in = [sem_a, sem_b]

    xcp = [
        pltpu.async_copy(xf_hbm.at[pl.ds(w * CT, SUB)], xin_v.at[0], sem_a),
        pltpu.async_copy(xf_hbm.at[pl.ds(w * CT + SUB, SUB)], xin_v.at[1],
                         sem_b),
    ]

    pltpu.sync_copy(cnts_hbm, cnt_v)
    pltpu.sync_copy(i1_hbm.at[pl.ds(w * CT, CT)], i1_v)
    pltpu.sync_copy(i2_hbm.at[pl.ds(w * CT, CT)], i2_v)

    zeros16 = jnp.zeros((16,), jnp.int32)
    ones16 = jnp.ones((16,), jnp.int32)
    lane = jax.lax.broadcasted_iota(jnp.int32, (16,), 0)
    wv = jnp.broadcast_to(w, (16,))

    tot = zeros16
    bef = zeros16
    for t in range(N // CT):
        row = cnt_v[t, 0:16]
        tot = tot + row
        bef = bef + jnp.where(jnp.full((16,), t, jnp.int32) < wv, row, zeros16)

    seg = (tot + (TILE - 1)) & jnp.int32(~(TILE - 1))
    po = plsc.cumsum(seg) - seg
    base = po + bef

    po_scalars = [jnp.sum(jnp.where(lane == e, po, zeros16)) for e in range(E)]
    base_vecs = [
        jnp.broadcast_to(jnp.sum(jnp.where(lane == e, base, zeros16)), (16,))
        for e in range(E)
    ]

    @pl.when(w == 0)
    def _():
        for g in range(NTP // 16):
            ts = (jax.lax.broadcasted_iota(jnp.int32, (16,), 0) + g * 16) * TILE
            acc = jnp.full((16,), -1, jnp.int32)
            for e in range(E):
                acc = acc + jnp.where(
                    ts >= jnp.broadcast_to(po_scalars[e], (16,)), 1, 0)
            texp_v[pl.ds(g * 16, 16)] = jnp.clip(acc, 0, E - 1)
        pltpu.sync_copy(texp_v, texp_hbm)

    cnts_run = [zeros16 for _ in range(E)]

    def process(src_v, dst_v):
        for j in range(CT // 16):
            v = src_v[pl.ds(j * 16, 16)]
            dest = zeros16
            for e in range(E):
                m = v == e
                r = plsc.cumsum(jnp.where(m, ones16, zeros16)) - 1
                dest = jnp.where(m, base_vecs[e] + cnts_run[e] + r, dest)
                cnts_run[e] = cnts_run[e] + plsc.all_reduce_population_count(m)
            dst_v[j // 4, pl.ds((j % 4) * 16, 16)] = dest

    process(i1_v, d0_v)
    process(i2_v, d1_v)

    pltpu.sync_copy(d0_v, d0_hbm.at[w])
    pltpu.sync_copy(d1_v, d1_hbm.at[w])

    sc_flight = [None, None]
    for sub in range(CT // SUB):
        bi = sub % 2
        xcp[bi].wait()
        sc_flight[bi] = (
            pltpu.async_copy(xin_v.at[bi], xs_hbm.at[d0_v.at[sub]],
                             sem_in[bi]),
            pltpu.async_copy(xin_v.at[bi], xs_hbm.at[d1_v.at[sub]],
                             sem_in[bi]),
        )
        for h in sc_flight[bi]:
            h.wait()
        sc_flight[bi] = None
        if sub + 2 < CT // SUB:
            xcp[bi] = pltpu.async_copy(
                xf_hbm.at[pl.ds(w * CT + (sub + 2) * SUB, SUB)],
                xin_v.at[bi], sem_in[bi])


SUBC = 32
NSUBC = CT // SUBC


@functools.lru_cache(maxsize=1)
def _make_combine():
    mesh = plsc.VectorSubcoreMesh(core_axis_name="c", subcore_axis_name="s")
    return functools.partial(
        pl.kernel,
        mesh=mesh,
        compiler_params=pltpu.CompilerParams(needs_layout_passes=False),
        out_type=jax.ShapeDtypeStruct((N, D), jnp.float32),
        scratch_types=[
            pltpu.VMEM((NSUBC, SUBC), jnp.int32),
            pltpu.VMEM((NSUBC, SUBC), jnp.int32),
            pltpu.VMEM((CT,), jnp.float32),
            pltpu.VMEM((CT,), jnp.float32),
            pltpu.VMEM((2, SUBC, D // 2), jnp.int32),
            pltpu.VMEM((2, SUBC, D // 2), jnp.int32),
            pltpu.VMEM((2, SUBC, D), jnp.float32),
            pltpu.SemaphoreType.DMA,
            pltpu.SemaphoreType.DMA,
            pltpu.SemaphoreType.DMA,
            pltpu.SemaphoreType.DMA,
        ],
    )(_combine_body)


def _combine_body(xf_hbm, ys_hbm, d0_hbm, d1_hbm, p1_hbm, p2_hbm,
                  out_hbm, d0_v, d1_v, p1_v, p2_v, g0_v, g1_v, xin_v,
                  sem_in0, sem_in1, sem_out0, sem_out1):
    c = lax.axis_index("c")
    s = lax.axis_index("s")
    w = s * 2 + c
    sem_in = [sem_in0, sem_in1]
    sem_out = [sem_out0, sem_out1]

    pltpu.sync_copy(d0_hbm.at[w], d0_v)
    pltpu.sync_copy(d1_hbm.at[w], d1_v)
    pltpu.sync_copy(p1_hbm.at[pl.ds(w * CT, CT)], p1_v)
    pltpu.sync_copy(p2_hbm.at[pl.ds(w * CT, CT)], p2_v)

    def fire(sub):
        bi = sub % 2
        t0 = w * CT + sub * SUBC
        return (
            pltpu.async_copy(xf_hbm.at[pl.ds(t0, SUBC)], xin_v.at[bi],
                             sem_in[bi]),
            pltpu.async_copy(ys_hbm.at[d0_v.at[sub]], g0_v.at[bi], sem_in[bi]),
            pltpu.async_copy(ys_hbm.at[d1_v.at[sub]], g1_v.at[bi], sem_in[bi]),
        )

    in_flight = [None, None]
    out_flight = [None, None]
    in_flight[0] = fire(0)

    for sub in range(NSUBC):
        bi = sub % 2
        if sub + 1 < NSUBC:
            nbi = (sub + 1) % 2
            if out_flight[nbi] is not None:
                out_flight[nbi].wait()
                out_flight[nbi] = None
            in_flight[nbi] = fire(sub + 1)
        for h in in_flight[bi]:
            h.wait()

        def row_body(r, carry, _sub=sub, _bi=bi):
            iv = jnp.full((16,), _sub * SUBC, jnp.int32) + r
            a0 = plsc.load_gather(p1_v, [iv])
            a1 = plsc.load_gather(p2_v, [iv])
            for q in range(D // 32):
                v0 = plsc.bitcast(g0_v[_bi, r, pl.ds(q * 16, 16)],
                                  jnp.bfloat16)
                v1 = plsc.bitcast(g1_v[_bi, r, pl.ds(q * 16, 16)],
                                  jnp.bfloat16)
                ya, yb = plsc.unpack(v0, format=plsc.PackFormat.INTERLEAVED)
                za, zb = plsc.unpack(v1, format=plsc.PackFormat.INTERLEAVED)
                sl_a = pl.ds(q * 32, 16)
                sl_b = pl.ds(q * 32 + 16, 16)
                xin_v[_bi, r, sl_a] = xin_v[_bi, r, sl_a] + a0 * ya + a1 * za
                xin_v[_bi, r, sl_b] = xin_v[_bi, r, sl_b] + a0 * yb + a1 * zb
            return carry

        lax.fori_loop(0, SUBC, row_body, 0)
        t0 = w * CT + sub * SUBC
        out_flight[bi] = pltpu.async_copy(
            xin_v.at[bi], out_hbm.at[pl.ds(t0, SUBC)], sem_out[bi])

    for h in out_flight:
        if h is not None:
            h.wait()


def _ffn_body(texp_ref, xs_ref, w1_ref, b1_ref, w2_ref, b2_ref, a_ref, ys_ref):
    i = pl.program_id(0)
    a = a_ref[texp_ref[i]]
    xb = xs_ref[...].astype(jnp.bfloat16)
    h = lax.dot_general(xb, w1_ref[0], (((1,), (1,)), ((), ())),
                        preferred_element_type=jnp.float32) + b1_ref[0]
    h = jnp.where(h > 0, h, a * h)
    y = lax.dot_general(h.astype(jnp.bfloat16), w2_ref[0],
                        (((1,), (1,)), ((), ())),
                        preferred_element_type=jnp.float32) + b2_ref[0]
    ys_ref[...] = y.astype(jnp.bfloat16)


def _ffn(xs, texp, w1b, b1r, w2b, b2r, a):
    grid_spec = pltpu.PrefetchScalarGridSpec(
        num_scalar_prefetch=1,
        grid=(NT,),
        in_specs=[
            pl.BlockSpec((TILE, D), lambda i, t: (i, 0)),
            pl.BlockSpec((1, D, D), lambda i, t: (t[i], 0, 0)),
            pl.BlockSpec((1, 1, D), lambda i, t: (t[i], 0, 0)),
            pl.BlockSpec((1, D, D), lambda i, t: (t[i], 0, 0)),
            pl.BlockSpec((1, 1, D), lambda i, t: (t[i], 0, 0)),
            pl.BlockSpec(memory_space=pltpu.SMEM),
        ],
        out_specs=pl.BlockSpec((TILE, D), lambda i, t: (i, 0)),
    )
    return pl.pallas_call(
        _ffn_body,
        grid_spec=grid_spec,
        out_shape=jax.ShapeDtypeStruct((P, D), jnp.bfloat16),
    )(texp, xs, w1b, b1r, w2b, b2r, a)


def kernel(x, gate_w, gate_b, w1, b1, a, w2, b2):
    b, c, t, h, w = x.shape
    xt = x.reshape(D, N)

    i1, i2, p1, p2, counts, xf = _gate(xt, gate_w, gate_b)
    d0, d1, texp, xs = _make_dispatch()(i1, i2, counts, xf)

    pos = jnp.arange(D, dtype=jnp.int32)
    k32, r32 = pos // 32, pos % 32
    sigma = 32 * k32 + jnp.where(r32 % 2 == 0, r32 // 2, 16 + r32 // 2)
    w1b = w1.astype(jnp.bfloat16)
    w2b = w2.astype(jnp.bfloat16)[:, sigma, :]
    b1r = b1.reshape(E, 1, D)
    b2r = b2[:, sigma].reshape(E, 1, D)

    ys = _ffn(xs, texp[:NT], w1b, b1r, w2b, b2r, a)
    ys = lax.bitcast_convert_type(ys.reshape(P, D // 2, 2), jnp.int32)

    d0c = d0.reshape(N // CT, NSUBC, SUBC)
    d1c = d1.reshape(N // CT, NSUBC, SUBC)
    outf = _make_combine()(xf, ys, d0c, d1c, p1, p2)
    out = outf.reshape(b, t, h, w, c)
    return jnp.transpose(out, (0, 4, 1, 2, 3))


# speedup vs baseline: 2.2209x; 2.2209x over previous
import functools

import jax
import jax.numpy as jnp
from jax import lax
from jax.experimental import pallas as pl
from jax.experimental.pallas import tpu as pltpu
from jax.experimental.pallas import tpu_sc as plsc

N = 8192
D = 512
E = 8
K = 2
A = N * K
TILE = 512
P = A + E * TILE
NT = P // TILE
NTP = 48
GT = 1024
LANES = 128
CT = 256
SUB = 64


def _gate_body(xt_ref, gw_ref, gb_ref,
               i1_ref, i2_ref, p1_ref, p2_ref, cnt_ref, xf_ref):
    xb = xt_ref[...]
    logits = lax.dot_general(gw_ref[...], xb, (((1,), (0,)), ((), ())),
                             preferred_element_type=jnp.float32,
                             precision=jax.lax.Precision.HIGHEST) + gb_ref[...]
    sub = jax.lax.broadcasted_iota(jnp.int32, (LANES, GT), 0)
    neg = jnp.float32(-1e30)
    l1 = jnp.where(sub < E, logits, neg)
    m1 = jnp.max(l1, axis=0)
    i1 = jnp.min(jnp.where(l1 == m1[None, :], sub, LANES), axis=0)
    l2 = jnp.where(sub == i1[None, :], neg, l1)
    m2 = jnp.max(l2, axis=0)
    i2 = jnp.min(jnp.where(l2 == m2[None, :], sub, LANES), axis=0)
    p1 = 1.0 / (1.0 + jnp.exp(m2 - m1))
    i1_ref[0, 0, :] = i1
    i2_ref[0, 0, :] = i2
    p1_ref[0, 0, :] = p1
    p2_ref[0, 0, :] = 1.0 - p1
    oh = ((sub == i1[None, :]).astype(jnp.int32)
          + (sub == i2[None, :]).astype(jnp.int32))
    for q in range(GT // CT):
        cnt_ref[0, q, :] = jnp.sum(oh[:, q * CT:(q + 1) * CT], axis=1)
    xf_ref[...] = xb.T


def _gate(xt, gate_w, gate_b):
    gwp = jnp.zeros((LANES, D), jnp.float32).at[:E].set(gate_w)
    gbp = jnp.zeros((LANES, 1), jnp.float32).at[:E, 0].set(gate_b)
    ng = N // GT
    nq = GT // CT
    outs = pl.pallas_call(
        _gate_body,
        grid=(ng,),
        in_specs=[
            pl.BlockSpec((D, GT), lambda i: (0, i)),
            pl.BlockSpec((LANES, D), lambda i: (0, 0)),
            pl.BlockSpec((LANES, 1), lambda i: (0, 0)),
        ],
        out_specs=[pl.BlockSpec((1, 1, GT), lambda i: (i, 0, 0))] * 4
        + [pl.BlockSpec((1, nq, LANES), lambda i: (i, 0, 0)),
           pl.BlockSpec((GT, D), lambda i: (i, 0))],
        out_shape=[
            jax.ShapeDtypeStruct((ng, 1, GT), jnp.int32),
            jax.ShapeDtypeStruct((ng, 1, GT), jnp.int32),
            jax.ShapeDtypeStruct((ng, 1, GT), jnp.float32),
            jax.ShapeDtypeStruct((ng, 1, GT), jnp.float32),
            jax.ShapeDtypeStruct((ng, nq, LANES), jnp.int32),
            jax.ShapeDtypeStruct((N, D), jnp.float32),
        ],
    )(xt, gwp, gbp)
    i1, i2, p1, p2 = (o.reshape(N) for o in outs[:4])
    counts = outs[4].reshape(N // CT, LANES)
    return i1, i2, p1, p2, counts, outs[5]


@functools.lru_cache(maxsize=1)
def _make_dispatch():
    mesh = plsc.VectorSubcoreMesh(core_axis_name="c", subcore_axis_name="s")
    return functools.partial(
        pl.kernel,
        mesh=mesh,
        compiler_params=pltpu.CompilerParams(needs_layout_passes=False),
        out_type=[
            jax.ShapeDtypeStruct((N // CT, CT // SUB, SUB), jnp.int32),
            jax.ShapeDtypeStruct((N // CT, CT // SUB, SUB), jnp.int32),
            jax.ShapeDtypeStruct((NTP,), jnp.int32),
            jax.ShapeDtypeStruct((P, D), jnp.float32),
        ],
        scratch_types=[
            pltpu.VMEM((N // CT, LANES), jnp.int32),
            pltpu.VMEM((CT,), jnp.int32),
            pltpu.VMEM((CT,), jnp.int32),
            pltpu.VMEM((CT // SUB, SUB), jnp.int32),
            pltpu.VMEM((CT // SUB, SUB), jnp.int32),
            pltpu.VMEM((NTP,), jnp.int32),
            pltpu.VMEM((2, SUB, D), jnp.float32),
            pltpu.SemaphoreType.DMA,
            pltpu.SemaphoreType.DMA,
        ],
    )(_dispatch_body)


def _dispatch_body(i1_hbm, i2_hbm, cnts_hbm, xf_hbm,
                   d0_hbm, d1_hbm, texp_hbm, xs_hbm,
                   cnt_v, i1_v, i2_v, d0_v, d1_v, texp_v, xin_v,
                   sem_a, sem_b):
    c = lax.axis_index("c")
    s = lax.axis_index("s")
    w = s * 2 + c
    sem_in = [sem_a, sem_b]

    xcp = [
        pltpu.async_copy(xf_hbm.at[pl.ds(w * CT, SUB)], xin_v.at[0], sem_a),
        pltpu.async_copy(xf_hbm.at[pl.ds(w * CT + SUB, SUB)], xin_v.at[1],
                         sem_b),
    ]

    pltpu.sync_copy(cnts_hbm, cnt_v)
    pltpu.sync_copy(i1_hbm.at[pl.ds(w * CT, CT)], i1_v)
    pltpu.sync_copy(i2_hbm.at[pl.ds(w * CT, CT)], i2_v)

    zeros16 = jnp.zeros((16,), jnp.int32)
    ones16 = jnp.ones((16,), jnp.int32)
    lane = jax.lax.broadcasted_iota(jnp.int32, (16,), 0)
    wv = jnp.broadcast_to(w, (16,))

    tot = zeros16
    bef = zeros16
    for t in range(N // CT):
        row = cnt_v[t, 0:16]
        tot = tot + row
        bef = bef + jnp.where(jnp.full((16,), t, jnp.int32) < wv, row, zeros16)

    seg = (tot + (TILE - 1)) & jnp.int32(~(TILE - 1))
    po = plsc.cumsum(seg) - seg
    base = po + bef

    po_scalars = [jnp.sum(jnp.where(lane == e, po, zeros16)) for e in range(E)]
    base_vecs = [
        jnp.broadcast_to(jnp.sum(jnp.where(lane == e, base, zeros16)), (16,))
        for e in range(E)
    ]

    @pl.when(w == 0)
    def _():
        for g in range(NTP // 16):
            ts = (jax.lax.broadcasted_iota(jnp.int32, (16,), 0) + g * 16) * TILE
            acc = jnp.full((16,), -1, jnp.int32)
            for e in range(E):
                acc = acc + jnp.where(
                    ts >= jnp.broadcast_to(po_scalars[e], (16,)), 1, 0)
            texp_v[pl.ds(g * 16, 16)] = jnp.clip(acc, 0, E - 1)
        pltpu.sync_copy(texp_v, texp_hbm)

    cnts_run = [zeros16 for _ in range(E)]

    def process(src_v, dst_v):
        for j in range(CT // 16):
            v = src_v[pl.ds(j * 16, 16)]
            dest = zeros16
            for e in range(E):
                m = v == e
                r = plsc.cumsum(jnp.where(m, ones16, zeros16)) - 1
                dest = jnp.where(m, base_vecs[e] + cnts_run[e] + r, dest)
                cnts_run[e] = cnts_run[e] + plsc.all_reduce_population_count(m)
            dst_v[j // 4, pl.ds((j % 4) * 16, 16)] = dest

    process(i1_v, d0_v)
    process(i2_v, d1_v)

    pltpu.sync_copy(d0_v, d0_hbm.at[w])
    pltpu.sync_copy(d1_v, d1_hbm.at[w])

    sc_flight = [None, None]
    for sub in range(CT // SUB):
        bi = sub % 2
        xcp[bi].wait()
        sc_flight[bi] = (
            pltpu.async_copy(xin_v.at[bi], xs_hbm.at[d0_v.at[sub]],
                             sem_in[bi]),
            pltpu.async_copy(xin_v.at[bi], xs_hbm.at[d1_v.at[sub]],
                             sem_in[bi]),
        )
        for h in sc_flight[bi]:
            h.wait()
        sc_flight[bi] = None
        if sub + 2 < CT // SUB:
            xcp[bi] = pltpu.async_copy(
                xf_hbm.at[pl.ds(w * CT + (sub + 2) * SUB, SUB)],
                xin_v.at[bi], sem_in[bi])


SUBC = 32
NSUBC = CT // SUBC


@functools.lru_cache(maxsize=1)
def _make_combine():
    mesh = plsc.VectorSubcoreMesh(core_axis_name="c", subcore_axis_name="s")
    return functools.partial(
        pl.kernel,
        mesh=mesh,
        compiler_params=pltpu.CompilerParams(needs_layout_passes=False),
        out_type=jax.ShapeDtypeStruct((N, D), jnp.float32),
        scratch_types=[
            pltpu.VMEM((NSUBC, SUBC), jnp.int32),
            pltpu.VMEM((NSUBC, SUBC), jnp.int32),
            pltpu.VMEM((CT,), jnp.float32),
            pltpu.VMEM((CT,), jnp.float32),
            pltpu.VMEM((2, SUBC, D // 2), jnp.int32),
            pltpu.VMEM((2, SUBC, D // 2), jnp.int32),
            pltpu.VMEM((2, SUBC, D), jnp.float32),
            pltpu.SemaphoreType.DMA,
            pltpu.SemaphoreType.DMA,
            pltpu.SemaphoreType.DMA,
            pltpu.SemaphoreType.DMA,
        ],
    )(_combine_body)


def _combine_body(xf_hbm, ys_hbm, d0_hbm, d1_hbm, p1_hbm, p2_hbm,
                  out_hbm, d0_v, d1_v, p1_v, p2_v, g0_v, g1_v, xin_v,
                  sem_in0, sem_in1, sem_out0, sem_out1):
    c = lax.axis_index("c")
    s = lax.axis_index("s")
    w = s * 2 + c
    sem_in = [sem_in0, sem_in1]
    sem_out = [sem_out0, sem_out1]

    pltpu.sync_copy(d0_hbm.at[w], d0_v)
    pltpu.sync_copy(d1_hbm.at[w], d1_v)
    pltpu.sync_copy(p1_hbm.at[pl.ds(w * CT, CT)], p1_v)
    pltpu.sync_copy(p2_hbm.at[pl.ds(w * CT, CT)], p2_v)

    def fire(sub):
        bi = sub % 2
        t0 = w * CT + sub * SUBC
        return (
            pltpu.async_copy(xf_hbm.at[pl.ds(t0, SUBC)], xin_v.at[bi],
                             sem_in[bi]),
            pltpu.async_copy(ys_hbm.at[d0_v.at[sub]], g0_v.at[bi], sem_in[bi]),
            pltpu.async_copy(ys_hbm.at[d1_v.at[sub]], g1_v.at[bi], sem_in[bi]),
        )

    in_flight = [None, None]
    out_flight = [None, None]
    in_flight[0] = fire(0)

    for sub in range(NSUBC):
        bi = sub % 2
        if sub + 1 < NSUBC:
            nbi = (sub + 1) % 2
            if out_flight[nbi] is not None:
                out_flight[nbi].wait()
                out_flight[nbi] = None
            in_flight[nbi] = fire(sub + 1)
        for h in in_flight[bi]:
            h.wait()

        def row_body(r, carry, _sub=sub, _bi=bi):
            iv = jnp.full((16,), _sub * SUBC, jnp.int32) + r
            a0 = plsc.load_gather(p1_v, [iv])
            a1 = plsc.load_gather(p2_v, [iv])
            for q in range(D // 32):
                v0 = plsc.bitcast(g0_v[_bi, r, pl.ds(q * 16, 16)],
                                  jnp.bfloat16)
                v1 = plsc.bitcast(g1_v[_bi, r, pl.ds(q * 16, 16)],
                                  jnp.bfloat16)
                ya, yb = plsc.unpack(v0, format=plsc.PackFormat.INTERLEAVED)
                za, zb = plsc.unpack(v1, format=plsc.PackFormat.INTERLEAVED)
                sl_a = pl.ds(q * 32, 16)
                sl_b = pl.ds(q * 32 + 16, 16)
                xin_v[_bi, r, sl_a] = xin_v[_bi, r, sl_a] + a0 * ya + a1 * za
                xin_v[_bi, r, sl_b] = xin_v[_bi, r, sl_b] + a0 * yb + a1 * zb
            return carry

        lax.fori_loop(0, SUBC, row_body, 0)
        t0 = w * CT + sub * SUBC
        out_flight[bi] = pltpu.async_copy(
            xin_v.at[bi], out_hbm.at[pl.ds(t0, SUBC)], sem_out[bi])

    for h in out_flight:
        if h is not None:
            h.wait()


def _ffn_body(texp_ref, xs_ref, w1_ref, b1_ref, w2_ref, b2_ref, a_ref, ys_ref):
    i = pl.program_id(0)
    a = a_ref[texp_ref[i]]
    xb = xs_ref[...].astype(jnp.bfloat16)
    h = lax.dot_general(xb, w1_ref[0], (((1,), (1,)), ((), ())),
                        preferred_element_type=jnp.float32) + b1_ref[0]
    h = jnp.where(h > 0, h, a * h)
    y = lax.dot_general(h.astype(jnp.bfloat16), w2_ref[0],
                        (((1,), (1,)), ((), ())),
                        preferred_element_type=jnp.float32) + b2_ref[0]
    yb = y.astype(jnp.bfloat16)
    lo = lax.bitcast_convert_type(yb[:, :D // 2], jnp.uint16)
    hi = lax.bitcast_convert_type(yb[:, D // 2:], jnp.uint16)
    word = (lo.astype(jnp.uint32)
            | (hi.astype(jnp.uint32) << jnp.uint32(16)))
    ys_ref[...] = lax.bitcast_convert_type(word, jnp.int32)


def _ffn(xs, texp, w1b, b1r, w2b, b2r, a):
    grid_spec = pltpu.PrefetchScalarGridSpec(
        num_scalar_prefetch=1,
        grid=(NT,),
        in_specs=[
            pl.BlockSpec((TILE, D), lambda i, t: (i, 0)),
            pl.BlockSpec((1, D, D), lambda i, t: (t[i], 0, 0)),
            pl.BlockSpec((1, 1, D), lambda i, t: (t[i], 0, 0)),
            pl.BlockSpec((1, D, D), lambda i, t: (t[i], 0, 0)),
            pl.BlockSpec((1, 1, D), lambda i, t: (t[i], 0, 0)),
            pl.BlockSpec(memory_space=pltpu.SMEM),
        ],
        out_specs=pl.BlockSpec((TILE, D // 2), lambda i, t: (i, 0)),
    )
    return pl.pallas_call(
        _ffn_body,
        grid_spec=grid_spec,
        out_shape=jax.ShapeDtypeStruct((P, D // 2), jnp.int32),
    )(texp, xs, w1b, b1r, w2b, b2r, a)


def kernel(x, gate_w, gate_b, w1, b1, a, w2, b2):
    b, c, t, h, w = x.shape
    xt = x.reshape(D, N)

    i1, i2, p1, p2, counts, xf = _gate(xt, gate_w, gate_b)
    d0, d1, texp, xs = _make_dispatch()(i1, i2, counts, xf)

    j = jnp.arange(D // 2, dtype=jnp.int32)
    q16, i16 = j // 16, j % 16
    sigma = jnp.concatenate([32 * q16 + i16, 32 * q16 + 16 + i16])
    w1b = w1.astype(jnp.bfloat16)
    w2b = w2.astype(jnp.bfloat16)[:, sigma, :]
    b1r = b1.reshape(E, 1, D)
    b2r = b2[:, sigma].reshape(E, 1, D)

    ys = _ffn(xs, texp[:NT], w1b, b1r, w2b, b2r, a)

    d0c = d0.reshape(N // CT, NSUBC, SUBC)
    d1c = d1.reshape(N // CT, NSUBC, SUBC)
    outf = _make_combine()(xf, ys, d0c, d1c, p1, p2)
    out = outf.reshape(b, t, h, w, c)
    return jnp.transpose(out, (0, 4, 1, 2, 3))


# speedup vs baseline: 2.4320x; 1.0951x over previous
import functools

import jax
import jax.numpy as jnp
from jax import lax
from jax.experimental import pallas as pl
from jax.experimental.pallas import tpu as pltpu
from jax.experimental.pallas import tpu_sc as plsc

N = 8192
D = 512
E = 8
K = 2
A = N * K
TILE = 512
P = A + E * TILE
NT = P // TILE
NTP = 48
GT = 1024
LANES = 128
CT = 256
SUB = 64


def _gate_body(xt_ref, gw_ref, gb_ref,
               i1_ref, i2_ref, p1_ref, p2_ref, cnt_ref, xf_ref):
    xb = xt_ref[...]
    logits = lax.dot_general(gw_ref[...], xb, (((1,), (0,)), ((), ())),
                             preferred_element_type=jnp.float32,
                             precision=jax.lax.Precision.HIGHEST) + gb_ref[...]
    sub = jax.lax.broadcasted_iota(jnp.int32, (LANES, GT), 0)
    neg = jnp.float32(-1e30)
    l1 = jnp.where(sub < E, logits, neg)
    m1 = jnp.max(l1, axis=0)
    i1 = jnp.min(jnp.where(l1 == m1[None, :], sub, LANES), axis=0)
    l2 = jnp.where(sub == i1[None, :], neg, l1)
    m2 = jnp.max(l2, axis=0)
    i2 = jnp.min(jnp.where(l2 == m2[None, :], sub, LANES), axis=0)
    p1 = 1.0 / (1.0 + jnp.exp(m2 - m1))
    i1_ref[0, 0, :] = i1
    i2_ref[0, 0, :] = i2
    p1_ref[0, 0, :] = p1
    p2_ref[0, 0, :] = 1.0 - p1
    oh = ((sub == i1[None, :]).astype(jnp.int32)
          + (sub == i2[None, :]).astype(jnp.int32))
    for q in range(GT // CT):
        cnt_ref[0, q, :] = jnp.sum(oh[:, q * CT:(q + 1) * CT], axis=1)
    xf_ref[...] = xb.T


def _gate(xt, gate_w, gate_b):
    gwp = jnp.zeros((LANES, D), jnp.float32).at[:E].set(gate_w)
    gbp = jnp.zeros((LANES, 1), jnp.float32).at[:E, 0].set(gate_b)
    ng = N // GT
    nq = GT // CT
    outs = pl.pallas_call(
        _gate_body,
        grid=(ng,),
        in_specs=[
            pl.BlockSpec((D, GT), lambda i: (0, i)),
            pl.BlockSpec((LANES, D), lambda i: (0, 0)),
            pl.BlockSpec((LANES, 1), lambda i: (0, 0)),
        ],
        out_specs=[pl.BlockSpec((1, 1, GT), lambda i: (i, 0, 0))] * 4
        + [pl.BlockSpec((1, nq, LANES), lambda i: (i, 0, 0)),
           pl.BlockSpec((GT, D), lambda i: (i, 0))],
        out_shape=[
            jax.ShapeDtypeStruct((ng, 1, GT), jnp.int32),
            jax.ShapeDtypeStruct((ng, 1, GT), jnp.int32),
            jax.ShapeDtypeStruct((ng, 1, GT), jnp.float32),
            jax.ShapeDtypeStruct((ng, 1, GT), jnp.float32),
            jax.ShapeDtypeStruct((ng, nq, LANES), jnp.int32),
            jax.ShapeDtypeStruct((N, D), jnp.float32),
        ],
    )(xt, gwp, gbp)
    i1, i2, p1, p2 = (o.reshape(N) for o in outs[:4])
    counts = outs[4].reshape(N // CT, LANES)
    return i1, i2, p1, p2, counts, outs[5]


@functools.lru_cache(maxsize=1)
def _make_dispatch():
    mesh = plsc.VectorSubcoreMesh(core_axis_name="c", subcore_axis_name="s")
    return functools.partial(
        pl.kernel,
        mesh=mesh,
        compiler_params=pltpu.CompilerParams(needs_layout_passes=False),
        out_type=[
            jax.ShapeDtypeStruct((N // CT, CT // SUB, SUB), jnp.int32),
            jax.ShapeDtypeStruct((N // CT, CT // SUB, SUB), jnp.int32),
            jax.ShapeDtypeStruct((NTP,), jnp.int32),
            jax.ShapeDtypeStruct((P, D), jnp.float32),
        ],
        scratch_types=[
            pltpu.VMEM((N // CT, LANES), jnp.int32),
            pltpu.VMEM((CT,), jnp.int32),
            pltpu.VMEM((CT,), jnp.int32),
            pltpu.VMEM((CT // SUB, SUB), jnp.int32),
            pltpu.VMEM((CT // SUB, SUB), jnp.int32),
            pltpu.VMEM((NTP,), jnp.int32),
            pltpu.VMEM((2, SUB, D), jnp.float32),
            pltpu.SemaphoreType.DMA,
            pltpu.SemaphoreType.DMA,
        ],
    )(_dispatch_body)


def _dispatch_body(i1_hbm, i2_hbm, cnts_hbm, xf_hbm,
                   d0_hbm, d1_hbm, texp_hbm, xs_hbm,
                   cnt_v, i1_v, i2_v, d0_v, d1_v, texp_v, xin_v,
                   sem_a, sem_b):
    c = lax.axis_index("c")
    s = lax.axis_index("s")
    w = s * 2 + c
    sem_in = [sem_a, sem_b]

    xcp = [
        pltpu.async_copy(xf_hbm.at[pl.ds(w * CT, SUB)], xin_v.at[0], sem_a),
        pltpu.async_copy(xf_hbm.at[pl.ds(w * CT + SUB, SUB)], xin_v.at[1],
                         sem_b),
    ]

    pltpu.sync_copy(cnts_hbm, cnt_v)
    pltpu.sync_copy(i1_hbm.at[pl.ds(w * CT, CT)], i1_v)
    pltpu.sync_copy(i2_hbm.at[pl.ds(w * CT, CT)], i2_v)

    zeros16 = jnp.zeros((16,), jnp.int32)
    ones16 = jnp.ones((16,), jnp.int32)
    lane = jax.lax.broadcasted_iota(jnp.int32, (16,), 0)
    wv = jnp.broadcast_to(w, (16,))

    tot = zeros16
    bef = zeros16
    for t in range(N // CT):
        row = cnt_v[t, 0:16]
        tot = tot + row
        bef = bef + jnp.where(jnp.full((16,), t, jnp.int32) < wv, row, zeros16)

    seg = (tot + (TILE - 1)) & jnp.int32(~(TILE - 1))
    po = plsc.cumsum(seg) - seg
    base = po + bef

    po_scalars = [jnp.sum(jnp.where(lane == e, po, zeros16)) for e in range(E)]
    base_vecs = [
        jnp.broadcast_to(jnp.sum(jnp.where(lane == e, base, zeros16)), (16,))
        for e in range(E)
    ]

    @pl.when(w == 0)
    def _():
        for g in range(NTP // 16):
            ts = (jax.lax.broadcasted_iota(jnp.int32, (16,), 0) + g * 16) * TILE
            acc = jnp.full((16,), -1, jnp.int32)
            for e in range(E):
                acc = acc + jnp.where(
                    ts >= jnp.broadcast_to(po_scalars[e], (16,)), 1, 0)
            texp_v[pl.ds(g * 16, 16)] = jnp.clip(acc, 0, E - 1)
        pltpu.sync_copy(texp_v, texp_hbm)

    cnts_run = [zeros16 for _ in range(E)]

    def process(src_v, dst_v):
        for j in range(CT // 16):
            v = src_v[pl.ds(j * 16, 16)]
            dest = zeros16
            for e in range(E):
                m = v == e
                r = plsc.cumsum(jnp.where(m, ones16, zeros16)) - 1
                dest = jnp.where(m, base_vecs[e] + cnts_run[e] + r, dest)
                cnts_run[e] = cnts_run[e] + plsc.all_reduce_population_count(m)
            dst_v[j // 4, pl.ds((j % 4) * 16, 16)] = dest

    process(i1_v, d0_v)
    process(i2_v, d1_v)

    pltpu.sync_copy(d0_v, d0_hbm.at[w])
    pltpu.sync_copy(d1_v, d1_hbm.at[w])

    sc_flight = [None, None]
    for sub in range(CT // SUB):
        bi = sub % 2
        xcp[bi].wait()
        sc_flight[bi] = (
            pltpu.async_copy(xin_v.at[bi], xs_hbm.at[d0_v.at[sub]],
                             sem_in[bi]),
            pltpu.async_copy(xin_v.at[bi], xs_hbm.at[d1_v.at[sub]],
                             sem_in[bi]),
        )
        for h in sc_flight[bi]:
            h.wait()
        sc_flight[bi] = None
        if sub + 2 < CT // SUB:
            xcp[bi] = pltpu.async_copy(
                xf_hbm.at[pl.ds(w * CT + (sub + 2) * SUB, SUB)],
                xin_v.at[bi], sem_in[bi])


SUBC = 32
NSUBC = CT // SUBC


@functools.lru_cache(maxsize=1)
def _make_combine():
    mesh = plsc.VectorSubcoreMesh(core_axis_name="c", subcore_axis_name="s")
    return functools.partial(
        pl.kernel,
        mesh=mesh,
        compiler_params=pltpu.CompilerParams(needs_layout_passes=False),
        out_type=jax.ShapeDtypeStruct((N, D), jnp.float32),
        scratch_types=[
            pltpu.VMEM((NSUBC, SUBC), jnp.int32),
            pltpu.VMEM((NSUBC, SUBC), jnp.int32),
            pltpu.VMEM((CT,), jnp.float32),
            pltpu.VMEM((CT,), jnp.float32),
            pltpu.VMEM((2, SUBC, D), jnp.float32),
            pltpu.VMEM((2, SUBC, D), jnp.float32),
            pltpu.VMEM((2, SUBC, D), jnp.float32),
            pltpu.SemaphoreType.DMA,
            pltpu.SemaphoreType.DMA,
            pltpu.SemaphoreType.DMA,
            pltpu.SemaphoreType.DMA,
        ],
    )(_combine_body)


def _combine_body(xf_hbm, ys_hbm, d0_hbm, d1_hbm, p1_hbm, p2_hbm,
                  out_hbm, d0_v, d1_v, p1_v, p2_v, g0_v, g1_v, xin_v,
                  sem_in0, sem_in1, sem_out0, sem_out1):
    c = lax.axis_index("c")
    s = lax.axis_index("s")
    w = s * 2 + c
    sem_in = [sem_in0, sem_in1]
    sem_out = [sem_out0, sem_out1]

    pltpu.sync_copy(d0_hbm.at[w], d0_v)
    pltpu.sync_copy(d1_hbm.at[w], d1_v)
    pltpu.sync_copy(p1_hbm.at[pl.ds(w * CT, CT)], p1_v)
    pltpu.sync_copy(p2_hbm.at[pl.ds(w * CT, CT)], p2_v)

    def fire(sub):
        bi = sub % 2
        t0 = w * CT + sub * SUBC
        return (
            pltpu.async_copy(xf_hbm.at[pl.ds(t0, SUBC)], xin_v.at[bi],
                             sem_in[bi]),
            pltpu.async_copy(ys_hbm.at[d0_v.at[sub]], g0_v.at[bi], sem_in[bi]),
            pltpu.async_copy(ys_hbm.at[d1_v.at[sub]], g1_v.at[bi], sem_in[bi]),
        )

    in_flight = [None, None]
    out_flight = [None, None]
    in_flight[0] = fire(0)

    for sub in range(NSUBC):
        bi = sub % 2
        if sub + 1 < NSUBC:
            nbi = (sub + 1) % 2
            if out_flight[nbi] is not None:
                out_flight[nbi].wait()
                out_flight[nbi] = None
            in_flight[nbi] = fire(sub + 1)
        for h in in_flight[bi]:
            h.wait()

        def row_body(r, carry, _sub=sub, _bi=bi):
            iv = jnp.full((16,), _sub * SUBC, jnp.int32) + r
            a0 = plsc.load_gather(p1_v, [iv])
            a1 = plsc.load_gather(p2_v, [iv])
            for q in range(D // 16):
                sl = pl.ds(q * 16, 16)
                xin_v[_bi, r, sl] = (xin_v[_bi, r, sl]
                                     + a0 * g0_v[_bi, r, sl]
                                     + a1 * g1_v[_bi, r, sl])
            return carry

        lax.fori_loop(0, SUBC, row_body, 0)
        t0 = w * CT + sub * SUBC
        out_flight[bi] = pltpu.async_copy(
            xin_v.at[bi], out_hbm.at[pl.ds(t0, SUBC)], sem_out[bi])

    for h in out_flight:
        if h is not None:
            h.wait()


def _ffn_body(texp_ref, xs_ref, w1_ref, b1_ref, w2_ref, b2_ref, a_ref, ys_ref):
    i = pl.program_id(0)
    a = a_ref[texp_ref[i]]
    xb = xs_ref[...].astype(jnp.bfloat16)
    h = lax.dot_general(xb, w1_ref[0], (((1,), (1,)), ((), ())),
                        preferred_element_type=jnp.float32) + b1_ref[0]
    h = jnp.where(h > 0, h, a * h)
    y = lax.dot_general(h.astype(jnp.bfloat16), w2_ref[0],
                        (((1,), (1,)), ((), ())),
                        preferred_element_type=jnp.float32) + b2_ref[0]
    ys_ref[...] = y


def _ffn(xs, texp, w1b, b1r, w2b, b2r, a):
    grid_spec = pltpu.PrefetchScalarGridSpec(
        num_scalar_prefetch=1,
        grid=(NT,),
        in_specs=[
            pl.BlockSpec((TILE, D), lambda i, t: (i, 0)),
            pl.BlockSpec((1, D, D), lambda i, t: (t[i], 0, 0)),
            pl.BlockSpec((1, 1, D), lambda i, t: (t[i], 0, 0)),
            pl.BlockSpec((1, D, D), lambda i, t: (t[i], 0, 0)),
            pl.BlockSpec((1, 1, D), lambda i, t: (t[i], 0, 0)),
            pl.BlockSpec(memory_space=pltpu.SMEM),
        ],
        out_specs=pl.BlockSpec((TILE, D), lambda i, t: (i, 0)),
    )
    return pl.pallas_call(
        _ffn_body,
        grid_spec=grid_spec,
        out_shape=jax.ShapeDtypeStruct((P, D), jnp.float32),
    )(texp, xs, w1b, b1r, w2b, b2r, a)


def kernel(x, gate_w, gate_b, w1, b1, a, w2, b2):
    b, c, t, h, w = x.shape
    xt = x.reshape(D, N)

    i1, i2, p1, p2, counts, xf = _gate(xt, gate_w, gate_b)
    d0, d1, texp, xs = _make_dispatch()(i1, i2, counts, xf)

    w1b = w1.astype(jnp.bfloat16)
    w2b = w2.astype(jnp.bfloat16)
    b1r = b1.reshape(E, 1, D)
    b2r = b2.reshape(E, 1, D)

    ys = _ffn(xs, texp[:NT], w1b, b1r, w2b, b2r, a)

    d0c = d0.reshape(N // CT, NSUBC, SUBC)
    d1c = d1.reshape(N // CT, NSUBC, SUBC)
    outf = _make_combine()(xf, ys, d0c, d1c, p1, p2)
    out = outf.reshape(b, t, h, w, c)
    return jnp.transpose(out, (0, 4, 1, 2, 3))
